# baseline (device time: 128286 ns/iter reference)
import jax
import jax.numpy as jnp
from jax import lax
from jax.experimental import pallas as pl
from jax.experimental.pallas import tpu as pltpu

N_DEV = 4


def kernel(A, B):
    m, k = A.shape
    _, n = B.shape
    chunk = m // N_DEV
    half = chunk // 2

    def top(c):
        return pl.ds(c * chunk, half)

    def bot(c):
        return pl.ds(c * chunk + half, half)

    f32 = jnp.float32
    bf16 = jnp.bfloat16

    L, R, D = 0, 1, 2
    TOP, BOT = 0, 1

    def body(a_hbm, b_ref, out_ref,
             a_tile, part, rs_recv,
             a_sem, rs_ssem, rs_rsem, ag_ssem, ag_rsem, ack_sem):
        my_pos = lax.axis_index("i")

        def at(pos):
            return lax.rem(my_pos + N_DEV + pos, N_DEV)

        left, right, diag = at(-1), at(1), at(2)

        barrier_sem = pltpu.get_barrier_semaphore()
        for nbr in (left, right, diag):
            pl.semaphore_signal(
                barrier_sem, inc=1,
                device_id=(nbr,), device_id_type=pl.DeviceIdType.MESH,
            )
        pl.semaphore_wait(barrier_sem, 3)

        def fetch_a(rows, slot):
            cp = pltpu.make_async_copy(
                a_hbm.at[rows, :],
                a_tile.at[slot],
                a_sem.at[slot],
            )
            cp.start()
            return cp

        def compute(rows, slot):
            part[rows, :] = jnp.dot(
                a_tile[slot], b_ref[:, :], preferred_element_type=f32,
            ).astype(bf16)

        sends = []

        def push_rs(rows, dest, peer_slot, half_slot, sem_idx):
            r = pltpu.make_async_remote_copy(
                src_ref=part.at[rows, :],
                dst_ref=rs_recv.at[peer_slot, half_slot],
                send_sem=rs_ssem.at[sem_idx],
                recv_sem=rs_rsem.at[peer_slot, half_slot],
                device_id=(dest,), device_id_type=pl.DeviceIdType.MESH,
            )
            r.start()
            sends.append(r)

        steps = [
            (top(at(0)), left, R, TOP, 0),
            (bot(at(0)), right, L, BOT, 1),
            (bot(at(1)), diag, D, BOT, 2),
            (top(at(-1)), diag, D, TOP, 3),
            (top(at(2)), right, L, TOP, 4),
            (bot(at(2)), left, R, BOT, 5),
            (top(at(1)), None, None, None, None),
            (bot(at(-1)), None, None, None, None),
        ]
        fetches = [fetch_a(steps[0][0], 0), fetch_a(steps[1][0], 1)]
        for i, (rows, dest, peer_slot, half_slot, sem_idx) in enumerate(steps):
            fetches[i % 2].wait()
            compute(rows, i % 2)
            if i + 2 < len(steps):
                fetches[i % 2] = fetch_a(steps[i + 2][0], i % 2)
            if dest is not None:
                push_rs(rows, dest, peer_slot, half_slot, sem_idx)

        def wait_rs(peer_slot, half_slot):
            pltpu.make_async_remote_copy(
                src_ref=part.at[top(0), :],
                dst_ref=rs_recv.at[peer_slot, half_slot],
                send_sem=rs_ssem.at[0],
                recv_sem=rs_rsem.at[peer_slot, half_slot],
                device_id=(left,), device_id_type=pl.DeviceIdType.MESH,
            ).wait_recv()

        def push_ag(rows, half_slot, dest, dst_peer, sem_idx):
            r = pltpu.make_async_remote_copy(
                src_ref=part.at[rows, :],
                dst_ref=rs_recv.at[dst_peer, half_slot],
                send_sem=ag_ssem.at[sem_idx],
                recv_sem=ag_rsem.at[dst_peer, half_slot],
                device_id=(dest,), device_id_type=pl.DeviceIdType.MESH,
            )
            r.start()
            sends.append(r)

        for p in (L, R, D):
            wait_rs(p, TOP)
        out_ref[top(at(1)), :] = jnp.maximum(
            part[top(at(1)), :].astype(f32)
            + rs_recv[L, TOP].astype(f32)
            + rs_recv[R, TOP].astype(f32)
            + rs_recv[D, TOP].astype(f32),
            0.0,
        )
        for p in (L, R, D):
            wait_rs(p, BOT)
        out_ref[bot(at(-1)), :] = jnp.maximum(
            part[bot(at(-1)), :].astype(f32)
            + rs_recv[L, BOT].astype(f32)
            + rs_recv[R, BOT].astype(f32)
            + rs_recv[D, BOT].astype(f32),
            0.0,
        )
        part[top(at(1)), :] = out_ref[top(at(1)), :].astype(bf16)
        part[bot(at(-1)), :] = out_ref[bot(at(-1)), :].astype(bf16)

        for nbr in (left, right, diag):
            pl.semaphore_signal(
                ack_sem, inc=1,
                device_id=(nbr,), device_id_type=pl.DeviceIdType.MESH,
            )
        pl.semaphore_wait(ack_sem, 3)

        push_ag(top(at(1)), TOP, left, R, 0)
        push_ag(top(at(1)), TOP, right, L, 1)
        push_ag(top(at(1)), TOP, diag, D, 2)
        push_ag(bot(at(-1)), BOT, left, R, 3)
        push_ag(bot(at(-1)), BOT, right, L, 4)
        push_ag(bot(at(-1)), BOT, diag, D, 5)

        def wait_ag(peer_slot, half_slot):
            pltpu.make_async_remote_copy(
                src_ref=part.at[top(0), :],
                dst_ref=rs_recv.at[peer_slot, half_slot],
                send_sem=ag_ssem.at[0],
                recv_sem=ag_rsem.at[peer_slot, half_slot],
                device_id=(left,), device_id_type=pl.DeviceIdType.MESH,
            ).wait_recv()

        wait_ag(L, TOP)
        out_ref[top(at(0)), :] = rs_recv[L, TOP].astype(f32)
        wait_ag(R, TOP)
        out_ref[top(at(2)), :] = rs_recv[R, TOP].astype(f32)
        wait_ag(D, TOP)
        out_ref[top(at(-1)), :] = rs_recv[D, TOP].astype(f32)
        wait_ag(L, BOT)
        out_ref[bot(at(2)), :] = rs_recv[L, BOT].astype(f32)
        wait_ag(R, BOT)
        out_ref[bot(at(0)), :] = rs_recv[R, BOT].astype(f32)
        wait_ag(D, BOT)
        out_ref[bot(at(1)), :] = rs_recv[D, BOT].astype(f32)

        for r in sends:
            r.wait_send()

    return pl.pallas_call(
        body,
        out_shape=jax.ShapeDtypeStruct((m, n), f32),
        in_specs=[
            pl.BlockSpec(memory_space=pltpu.MemorySpace.HBM),
            pl.BlockSpec(memory_space=pltpu.VMEM),
        ],
        out_specs=pl.BlockSpec(memory_space=pltpu.VMEM),
        scratch_shapes=[
            pltpu.VMEM((2, half, k), f32),
            pltpu.VMEM((m, n), bf16),
            pltpu.VMEM((3, 2, half, n), bf16),
            pltpu.SemaphoreType.DMA((2,)),
            pltpu.SemaphoreType.DMA((6,)),
            pltpu.SemaphoreType.DMA((3, 2)),
            pltpu.SemaphoreType.DMA((6,)),
            pltpu.SemaphoreType.DMA((3, 2)),
            pltpu.SemaphoreType.REGULAR,
        ],
        compiler_params=pltpu.CompilerParams(
            collective_id=0,
            vmem_limit_bytes=40 * 1024 * 1024,
        ),
    )(A, B)


# device time: 101181 ns/iter; 1.2679x vs baseline; 1.2679x over previous
import jax
import jax.numpy as jnp
from jax import lax
from jax.experimental import pallas as pl
from jax.experimental.pallas import tpu as pltpu

N_DEV = 4
UNITS = 2


def kernel(A, B):
    m, k = A.shape
    _, n = B.shape
    chunk = m // N_DEV
    half = chunk // 2
    qrt = half // UNITS

    f32 = jnp.float32
    bf16 = jnp.bfloat16

    def body(a_hbm, b_ref, out_ref,
             a_tile, stage_cw, recv_cw, stage_ccw, recv_ccw,
             a_sem, st_sem_cw, rc_sem_cw, st_sem_ccw, rc_sem_ccw):
        my_pos = lax.axis_index("i")

        def at(pos):
            return lax.rem(my_pos + N_DEV + pos, N_DEV)

        left, right = at(-1), at(1)

        barrier_sem = pltpu.get_barrier_semaphore()
        for nbr in (left, right):
            pl.semaphore_signal(
                barrier_sem, inc=1,
                device_id=(nbr,), device_id_type=pl.DeviceIdType.MESH,
            )
        pl.semaphore_wait(barrier_sem, 2)

        def rows_cw(c, u):
            return pl.ds(c * chunk + u * qrt, qrt)

        def rows_ccw(c, u):
            return pl.ds(c * chunk + half + u * qrt, qrt)

        CW = (rows_cw, right, stage_cw, recv_cw, st_sem_cw, rc_sem_cw)
        CCW = (rows_ccw, left, stage_ccw, recv_ccw, st_sem_ccw, rc_sem_ccw)

        def rdma(d, u, src_kind, src_slot, recv_slot):
            rows_fn, dest, stage, recv, st_sem, rc_sem = d
            if src_kind == "stage":
                src = stage.at[u, src_slot]
                ssem = st_sem.at[u, src_slot]
            else:
                src = recv.at[u, src_slot]
                ssem = st_sem.at[u, 3 - src_slot]
            return pltpu.make_async_remote_copy(
                src_ref=src,
                dst_ref=recv.at[u, recv_slot],
                send_sem=ssem,
                recv_sem=rc_sem.at[u, recv_slot],
                device_id=(dest,), device_id_type=pl.DeviceIdType.MESH,
            )

        sub_rows = [
            pl.ds(at(0) * chunk, half),
            pl.ds(at(0) * chunk + half, half),
            pl.ds(at(-1) * chunk, half),
            pl.ds(at(1) * chunk + half, half),
            pl.ds(at(2) * chunk, half),
            pl.ds(at(2) * chunk + half, half),
            pl.ds(at(1) * chunk, half),
            pl.ds(at(-1) * chunk + half, half),
        ]

        def fetch_a(i, slot):
            cp = pltpu.make_async_copy(
                a_hbm.at[sub_rows[i], :], a_tile.at[slot], a_sem.at[slot])
            cp.start()
            return cp

        def dot(i, slot):
            out_ref[sub_rows[i], :] = jnp.dot(
                a_tile[slot], b_ref[:, :], preferred_element_type=f32)

        fetches = [fetch_a(0, 0), fetch_a(1, 1)]

        def step_compute(i):
            fetches[i % 2].wait()
            dot(i, i % 2)
            if i + 2 < len(sub_rows):
                fetches[i % 2] = fetch_a(i + 2, i % 2)

        step_compute(0)
        h_cw = []
        for u in range(UNITS):
            stage_cw[u, 0] = out_ref[rows_cw(at(0), u), :].astype(bf16)
            r = rdma(CW, u, "stage", 0, 0)
            r.start()
            h_cw.append(r)
        step_compute(1)
        h_ccw = []
        for u in range(UNITS):
            stage_ccw[u, 0] = out_ref[rows_ccw(at(0), u), :].astype(bf16)
            r = rdma(CCW, u, "stage", 0, 0)
            r.start()
            h_ccw.append(r)

        step_compute(2)
        step_compute(3)

        for u in range(UNITS):
            h_cw[u].wait()
            stage_cw[u, 1] = (out_ref[rows_cw(at(-1), u), :]
                              + recv_cw[u, 0].astype(f32)).astype(bf16)
            h_cw[u] = rdma(CW, u, "stage", 1, 1)
            h_cw[u].start()
        for u in range(UNITS):
            h_ccw[u].wait()
            stage_ccw[u, 1] = (out_ref[rows_ccw(at(1), u), :]
                               + recv_ccw[u, 0].astype(f32)).astype(bf16)
            h_ccw[u] = rdma(CCW, u, "stage", 1, 1)
            h_ccw[u].start()

        step_compute(4)
        step_compute(5)

        for u in range(UNITS):
            h_cw[u].wait()
            stage_cw[u, 2] = (out_ref[rows_cw(at(2), u), :]
                              + recv_cw[u, 1].astype(f32)).astype(bf16)
            h_cw[u] = rdma(CW, u, "stage", 2, 0)
            h_cw[u].start()
        for u in range(UNITS):
            h_ccw[u].wait()
            stage_ccw[u, 2] = (out_ref[rows_ccw(at(2), u), :]
                               + recv_ccw[u, 1].astype(f32)).astype(bf16)
            h_ccw[u] = rdma(CCW, u, "stage", 2, 0)
            h_ccw[u].start()

        step_compute(6)
        step_compute(7)

        for u in range(UNITS):
            h_cw[u].wait()
            out_ref[rows_cw(at(1), u), :] = jnp.maximum(
                out_ref[rows_cw(at(1), u), :] + recv_cw[u, 0].astype(f32), 0.0)
            stage_cw[u, 0] = out_ref[rows_cw(at(1), u), :].astype(bf16)
            h_cw[u] = rdma(CW, u, "stage", 0, 2)
            h_cw[u].start()
        for u in range(UNITS):
            h_ccw[u].wait()
            out_ref[rows_ccw(at(-1), u), :] = jnp.maximum(
                out_ref[rows_ccw(at(-1), u), :] + recv_ccw[u, 0].astype(f32),
                0.0)
            stage_ccw[u, 0] = out_ref[rows_ccw(at(-1), u), :].astype(bf16)
            h_ccw[u] = rdma(CCW, u, "stage", 0, 2)
            h_ccw[u].start()

        for u in range(UNITS):
            h_cw[u].wait()
            h_cw[u] = rdma(CW, u, "recv", 2, 1)
            h_cw[u].start()
            out_ref[rows_cw(at(0), u), :] = recv_cw[u, 2].astype(f32)
        for u in range(UNITS):
            h_ccw[u].wait()
            h_ccw[u] = rdma(CCW, u, "recv", 2, 1)
            h_ccw[u].start()
            out_ref[rows_ccw(at(0), u), :] = recv_ccw[u, 2].astype(f32)

        for u in range(UNITS):
            h_cw[u].wait()
            h_cw[u] = rdma(CW, u, "recv", 1, 0)
            h_cw[u].start()
            out_ref[rows_cw(at(-1), u), :] = recv_cw[u, 1].astype(f32)
        for u in range(UNITS):
            h_ccw[u].wait()
            h_ccw[u] = rdma(CCW, u, "recv", 1, 0)
            h_ccw[u].start()
            out_ref[rows_ccw(at(1), u), :] = recv_ccw[u, 1].astype(f32)

        for u in range(UNITS):
            h_cw[u].wait()
            out_ref[rows_cw(at(2), u), :] = recv_cw[u, 0].astype(f32)
        for u in range(UNITS):
            h_ccw[u].wait()
            out_ref[rows_ccw(at(2), u), :] = recv_ccw[u, 0].astype(f32)

    comm = lambda: pltpu.VMEM((UNITS, 3, qrt, n), bf16)
    return pl.pallas_call(
        body,
        out_shape=jax.ShapeDtypeStruct((m, n), f32),
        in_specs=[
            pl.BlockSpec(memory_space=pltpu.MemorySpace.HBM),
            pl.BlockSpec(memory_space=pltpu.VMEM),
        ],
        out_specs=pl.BlockSpec(memory_space=pltpu.VMEM),
        scratch_shapes=[
            pltpu.VMEM((2, half, k), f32),
            comm(), comm(),
            comm(), comm(),
            pltpu.SemaphoreType.DMA((2,)),
            pltpu.SemaphoreType.DMA((UNITS, 3)),
            pltpu.SemaphoreType.DMA((UNITS, 3)),
            pltpu.SemaphoreType.DMA((UNITS, 3)),
            pltpu.SemaphoreType.DMA((UNITS, 3)),
        ],
        compiler_params=pltpu.CompilerParams(
            collective_id=0,
            vmem_limit_bytes=40 * 1024 * 1024,
        ),
    )(A, B)
